# Initial kernel scaffold; baseline (speedup 1.0000x reference)
import jax
import jax.numpy as jnp
from jax.experimental import pallas as pl


def kernel(inputs, sc_ind):
    raise NotImplementedError("write your pallas kernel here")


# speedup vs baseline: 1.3376x; 1.3376x over previous
import jax
import jax.numpy as jnp
from jax import lax
from jax.experimental import pallas as pl
from jax.experimental.pallas import tpu as pltpu, tpu_sc as plsc

_B, _T, _S, _O, _F = 16, 4, 2, 14, 4096
_NSC = 3276
_ROWS = _B * _T * _S * _O
_NW = 32
_ROWS_PER_W = _ROWS // _NW
_LANES = 16
_NSC_PAD = ((_NSC + _LANES - 1) // _LANES) * _LANES
_NJ = _NSC_PAD // _LANES


def _gather_body(in_hbm, idx_hbm, out_hbm, idx_v, row_v, out_v):
    c = lax.axis_index("c")
    s = lax.axis_index("s")
    wid = s * 2 + c
    pltpu.sync_copy(idx_hbm, idx_v)

    def row_body(i, carry):
        r = wid * _ROWS_PER_W + i
        pltpu.sync_copy(in_hbm.at[r], row_v)

        def j_body(j, carry2):
            idx = idx_v[pl.ds(j * _LANES, _LANES)]
            out_v[pl.ds(j * _LANES, _LANES)] = plsc.load_gather(row_v, [idx])
            return carry2

        lax.fori_loop(0, _NJ, j_body, 0)
        pltpu.sync_copy(out_v.at[pl.ds(0, _NSC)], out_hbm.at[r])
        return carry

    lax.fori_loop(0, _ROWS_PER_W, row_body, 0)


def kernel(inputs, sc_ind):
    x = inputs.reshape(_ROWS, _F)
    idx = jnp.pad(sc_ind.astype(jnp.int32), (0, _NSC_PAD - _NSC))
    mesh = plsc.VectorSubcoreMesh(core_axis_name="c", subcore_axis_name="s")
    out = pl.kernel(
        _gather_body,
        mesh=mesh,
        compiler_params=pltpu.CompilerParams(
            needs_layout_passes=False, use_tc_tiling_on_sc=False
        ),
        out_type=jax.ShapeDtypeStruct((_ROWS, _NSC), jnp.float32),
        scratch_types=[
            pltpu.VMEM((_NSC_PAD,), jnp.int32),
            pltpu.VMEM((_F,), jnp.float32),
            pltpu.VMEM((_NSC_PAD,), jnp.float32),
        ],
    )(x, idx)
    return out.reshape(_B, _T, _S, _O, _NSC)


# speedup vs baseline: 1.8260x; 1.3651x over previous
import jax
import jax.numpy as jnp
from jax import lax
from jax.experimental import pallas as pl
from jax.experimental.pallas import tpu as pltpu, tpu_sc as plsc

_B, _T, _S, _O, _F = 16, 4, 2, 14, 4096
_NSC = 3276
_ROWS = _B * _T * _S * _O
_NW = 32
_ROWS_PER_W = _ROWS // _NW
_LANES = 16
_NJ = (_NSC + _LANES - 1) // _LANES
_CH = 8
_NCHUNK = _ROWS_PER_W // _CH
_LAST_OFF = _NSC - _LANES


def _gather_body(in_hbm, idx_hbm, out_hbm, idx_v, in_v0, in_v1, out_v0,
                 out_v1, isem0, isem1, osem0, osem1):
    c = lax.axis_index("c")
    s = lax.axis_index("s")
    wid = s * 2 + c
    row0 = wid * _ROWS_PER_W
    pltpu.sync_copy(idx_hbm, idx_v)

    in_bufs = (in_v0, in_v1)
    out_bufs = (out_v0, out_v1)
    in_sems = (isem0, isem1)
    out_sems = (osem0, osem1)

    def start_in(ch):
        base = row0 + ch * _CH
        return pltpu.async_copy(
            in_hbm.at[pl.ds(base, _CH)], in_bufs[ch % 2], in_sems[ch % 2]
        )

    def start_out(ch):
        base = row0 + ch * _CH
        return pltpu.async_copy(
            out_bufs[ch % 2], out_hbm.at[pl.ds(base, _CH)], out_sems[ch % 2]
        )

    def gather_chunk(in_v, out_v):
        def j_body(j, carry):
            idx = idx_v[pl.ds(j * _LANES, _LANES)]
            off = jnp.where(j < _NJ - 1, j * _LANES, _LAST_OFF)
            for r in range(_CH):
                out_v[r, pl.ds(off, _LANES)] = plsc.load_gather(
                    in_v.at[r], [idx]
                )
            return carry

        lax.fori_loop(0, _NJ, j_body, 0)

    in_copies = [None] * _NCHUNK
    out_copies = [None] * _NCHUNK
    in_copies[0] = start_in(0)
    for ch in range(_NCHUNK):
        if ch + 1 < _NCHUNK:
            in_copies[ch + 1] = start_in(ch + 1)
        in_copies[ch].wait()
        if ch >= 2:
            out_copies[ch - 2].wait()
        gather_chunk(in_bufs[ch % 2], out_bufs[ch % 2])
        out_copies[ch] = start_out(ch)
    out_copies[_NCHUNK - 2].wait()
    out_copies[_NCHUNK - 1].wait()


def kernel(inputs, sc_ind):
    x = inputs.reshape(_ROWS, _F)
    idx32 = sc_ind.astype(jnp.int32)
    idx = jnp.concatenate(
        [idx32[: (_NJ - 1) * _LANES], idx32[_LAST_OFF:]]
    )
    mesh = plsc.VectorSubcoreMesh(core_axis_name="c", subcore_axis_name="s")
    out = pl.kernel(
        _gather_body,
        mesh=mesh,
        compiler_params=pltpu.CompilerParams(
            needs_layout_passes=False, use_tc_tiling_on_sc=False
        ),
        out_type=jax.ShapeDtypeStruct((_ROWS, _NSC), jnp.float32),
        scratch_types=[
            pltpu.VMEM((_NJ * _LANES,), jnp.int32),
            pltpu.VMEM((_CH, _F), jnp.float32),
            pltpu.VMEM((_CH, _F), jnp.float32),
            pltpu.VMEM((_CH, _NSC), jnp.float32),
            pltpu.VMEM((_CH, _NSC), jnp.float32),
            pltpu.SemaphoreType.DMA,
            pltpu.SemaphoreType.DMA,
            pltpu.SemaphoreType.DMA,
            pltpu.SemaphoreType.DMA,
        ],
    )(x, idx)
    return out.reshape(_B, _T, _S, _O, _NSC)


# speedup vs baseline: 2.1355x; 1.1695x over previous
import jax
import jax.numpy as jnp
from jax import lax
from jax.experimental import pallas as pl
from jax.experimental.pallas import tpu as pltpu, tpu_sc as plsc

_B, _T, _S, _O, _F = 16, 4, 2, 14, 4096
_NSC = 3276
_ROWS = _B * _T * _S * _O
_NW = 32
_ROWS_PER_W = _ROWS // _NW
_LANES = 16
_NJ = (_NSC + _LANES - 1) // _LANES
_CH = 8
_NCHUNK = _ROWS_PER_W // _CH
_LAST_OFF = _NSC - _LANES


def _gather_body(in_hbm, idx_hbm, out_hbm, idx_v, in_v0, in_v1, out_v0,
                 out_v1, isem0, isem1, osem0, osem1):
    c = lax.axis_index("c")
    s = lax.axis_index("s")
    wid = s * 2 + c
    row0 = wid * _ROWS_PER_W
    pltpu.sync_copy(idx_hbm, idx_v)

    in_bufs = (in_v0, in_v1)
    out_bufs = (out_v0, out_v1)
    in_sems = (isem0, isem1)
    out_sems = (osem0, osem1)

    def start_in(ch):
        base = row0 + ch * _CH
        return pltpu.async_copy(
            in_hbm.at[pl.ds(base, _CH)], in_bufs[ch % 2], in_sems[ch % 2]
        )

    def start_out(ch):
        base = row0 + ch * _CH
        return pltpu.async_copy(
            out_bufs[ch % 2], out_hbm.at[pl.ds(base, _CH)], out_sems[ch % 2]
        )

    def gather_chunk(in_v, out_v):
        @plsc.parallel_loop(0, _NJ, unroll=2)
        def j_body(j):
            idx = idx_v[pl.ds(j * _LANES, _LANES)]
            off = jnp.where(j < _NJ - 1, j * _LANES, _LAST_OFF)
            for r in range(_CH):
                out_v[r, pl.ds(off, _LANES)] = plsc.load_gather(
                    in_v.at[r], [idx]
                )

    in_copies = [None] * _NCHUNK
    out_copies = [None] * _NCHUNK
    in_copies[0] = start_in(0)
    for ch in range(_NCHUNK):
        if ch + 1 < _NCHUNK:
            in_copies[ch + 1] = start_in(ch + 1)
        in_copies[ch].wait()
        if ch >= 2:
            out_copies[ch - 2].wait()
        gather_chunk(in_bufs[ch % 2], out_bufs[ch % 2])
        out_copies[ch] = start_out(ch)
    out_copies[_NCHUNK - 2].wait()
    out_copies[_NCHUNK - 1].wait()


def kernel(inputs, sc_ind):
    x = inputs.reshape(_ROWS, _F)
    idx32 = sc_ind.astype(jnp.int32)
    idx = jnp.concatenate(
        [idx32[: (_NJ - 1) * _LANES], idx32[_LAST_OFF:]]
    )
    mesh = plsc.VectorSubcoreMesh(core_axis_name="c", subcore_axis_name="s")
    out = pl.kernel(
        _gather_body,
        mesh=mesh,
        compiler_params=pltpu.CompilerParams(
            needs_layout_passes=False, use_tc_tiling_on_sc=False
        ),
        out_type=jax.ShapeDtypeStruct((_ROWS, _NSC), jnp.float32),
        scratch_types=[
            pltpu.VMEM((_NJ * _LANES,), jnp.int32),
            pltpu.VMEM((_CH, _F), jnp.float32),
            pltpu.VMEM((_CH, _F), jnp.float32),
            pltpu.VMEM((_CH, _NSC), jnp.float32),
            pltpu.VMEM((_CH, _NSC), jnp.float32),
            pltpu.SemaphoreType.DMA,
            pltpu.SemaphoreType.DMA,
            pltpu.SemaphoreType.DMA,
            pltpu.SemaphoreType.DMA,
        ],
    )(x, idx)
    return out.reshape(_B, _T, _S, _O, _NSC)


# speedup vs baseline: 2.1962x; 1.0284x over previous
import jax
import jax.numpy as jnp
from jax import lax
from jax.experimental import pallas as pl
from jax.experimental.pallas import tpu as pltpu, tpu_sc as plsc

_B, _T, _S, _O, _F = 16, 4, 2, 14, 4096
_NSC = 3276
_NSLAB = _B * _T * _S
_NW = 32
_SLABS_PER_W = _NSLAB // _NW
_LANES = 16
_NJ = (_NSC + _LANES - 1) // _LANES
_CH = 7
_NCHUNK = 2 * _SLABS_PER_W
_LAST_OFF = _NSC - _LANES


def _gather_body(in_hbm, idx_hbm, out_hbm, idx_v, in_v0, in_v1, out_v0,
                 out_v1, isem0, isem1, osem0, osem1):
    c = lax.axis_index("c")
    s = lax.axis_index("s")
    wid = s * 2 + c
    slab0 = wid * _SLABS_PER_W
    pltpu.sync_copy(idx_hbm, idx_v)

    in_bufs = (in_v0, in_v1)
    out_bufs = (out_v0, out_v1)
    in_sems = (isem0, isem1)
    out_sems = (osem0, osem1)

    def slab_ref(hbm, ch):
        n = slab0 + ch // 2
        b = n // (_T * _S)
        t = (n // _S) % _T
        ss = n % _S
        return hbm.at[b, t, ss, pl.ds((ch % 2) * _CH, _CH)]

    def start_in(ch):
        return pltpu.async_copy(
            slab_ref(in_hbm, ch), in_bufs[ch % 2], in_sems[ch % 2]
        )

    def start_out(ch):
        return pltpu.async_copy(
            out_bufs[ch % 2], slab_ref(out_hbm, ch), out_sems[ch % 2]
        )

    def gather_chunk(in_v, out_v):
        @plsc.parallel_loop(0, _NJ, unroll=2)
        def j_body(j):
            idx = idx_v[pl.ds(j * _LANES, _LANES)]
            off = jnp.where(j < _NJ - 1, j * _LANES, _LAST_OFF)
            for r in range(_CH):
                out_v[r, pl.ds(off, _LANES)] = plsc.load_gather(
                    in_v.at[r], [idx]
                )

    in_copies = [None] * _NCHUNK
    out_copies = [None] * _NCHUNK
    in_copies[0] = start_in(0)
    for ch in range(_NCHUNK):
        if ch + 1 < _NCHUNK:
            in_copies[ch + 1] = start_in(ch + 1)
        in_copies[ch].wait()
        if ch >= 2:
            out_copies[ch - 2].wait()
        gather_chunk(in_bufs[ch % 2], out_bufs[ch % 2])
        out_copies[ch] = start_out(ch)
    out_copies[_NCHUNK - 2].wait()
    out_copies[_NCHUNK - 1].wait()


def kernel(inputs, sc_ind):
    idx32 = sc_ind.astype(jnp.int32)
    idx = jnp.concatenate(
        [idx32[: (_NJ - 1) * _LANES], idx32[_LAST_OFF:]]
    )
    mesh = plsc.VectorSubcoreMesh(core_axis_name="c", subcore_axis_name="s")
    out = pl.kernel(
        _gather_body,
        mesh=mesh,
        compiler_params=pltpu.CompilerParams(
            needs_layout_passes=False, use_tc_tiling_on_sc=False
        ),
        out_type=jax.ShapeDtypeStruct((_B, _T, _S, _O, _NSC), jnp.float32),
        scratch_types=[
            pltpu.VMEM((_NJ * _LANES,), jnp.int32),
            pltpu.VMEM((_CH, _F), jnp.float32),
            pltpu.VMEM((_CH, _F), jnp.float32),
            pltpu.VMEM((_CH, _NSC), jnp.float32),
            pltpu.VMEM((_CH, _NSC), jnp.float32),
            pltpu.SemaphoreType.DMA,
            pltpu.SemaphoreType.DMA,
            pltpu.SemaphoreType.DMA,
            pltpu.SemaphoreType.DMA,
        ],
    )(inputs, idx)
    return out


# speedup vs baseline: 2.3560x; 1.0728x over previous
import jax
import jax.numpy as jnp
from jax import lax
from jax.experimental import pallas as pl
from jax.experimental.pallas import tpu as pltpu, tpu_sc as plsc

_B, _T, _S, _O, _F = 16, 4, 2, 14, 4096
_NSC = 3276
_OP = 16
_NSCP = 3328
_NSLAB = _B * _T * _S
_NW = 32
_SLABS_PER_W = _NSLAB // _NW
_LANES = 16
_NJ = (_NSC + _LANES - 1) // _LANES
_CH = 7


def _gather_body(in_hbm, idx_hbm, out_hbm, idx_v, in_v0, in_v1, out_v,
                 isem0, isem1, osem):
    c = lax.axis_index("c")
    s = lax.axis_index("s")
    wid = s * 2 + c
    slab0 = wid * _SLABS_PER_W
    pltpu.sync_copy(idx_hbm, idx_v)

    in_bufs = (in_v0, in_v1)
    in_sems = (isem0, isem1)

    def slab_idx(n):
        return n // (_T * _S), (n // _S) % _T, n % _S

    def start_in(ch):
        b, t, ss = slab_idx(slab0 + ch // 2)
        return pltpu.async_copy(
            in_hbm.at[b, t, ss, pl.ds((ch % 2) * _CH, _CH)],
            in_bufs[ch % 2],
            in_sems[ch % 2],
        )

    def start_out(k):
        b, t, ss = slab_idx(slab0 + k)
        return pltpu.async_copy(out_v, out_hbm.at[b, t, ss], osem)

    def gather_chunk(in_v, row_base):
        @plsc.parallel_loop(0, _NJ, unroll=2)
        def j_body(j):
            idx = idx_v[pl.ds(j * _LANES, _LANES)]
            for r in range(_CH):
                out_v[row_base + r, pl.ds(j * _LANES, _LANES)] = (
                    plsc.load_gather(in_v.at[r], [idx])
                )

    in_copies = [None] * (2 * _SLABS_PER_W)
    out_copy = None
    in_copies[0] = start_in(0)
    for k in range(_SLABS_PER_W):
        for h in range(2):
            ch = 2 * k + h
            if ch + 1 < 2 * _SLABS_PER_W:
                in_copies[ch + 1] = start_in(ch + 1)
            in_copies[ch].wait()
            if h == 0 and out_copy is not None:
                out_copy.wait()
            gather_chunk(in_bufs[ch % 2], h * _CH)
        out_copy = start_out(k)
    out_copy.wait()


def kernel(inputs, sc_ind):
    idx32 = sc_ind.astype(jnp.int32)
    idx = jnp.concatenate(
        [idx32, jnp.full((_NJ * _LANES - _NSC,), idx32[-1], jnp.int32)]
    )
    mesh = plsc.VectorSubcoreMesh(core_axis_name="c", subcore_axis_name="s")
    out = pl.kernel(
        _gather_body,
        mesh=mesh,
        compiler_params=pltpu.CompilerParams(
            needs_layout_passes=False, use_tc_tiling_on_sc=False
        ),
        out_type=jax.ShapeDtypeStruct((_B, _T, _S, _OP, _NSCP), jnp.float32),
        scratch_types=[
            pltpu.VMEM((_NJ * _LANES,), jnp.int32),
            pltpu.VMEM((_CH, _F), jnp.float32),
            pltpu.VMEM((_CH, _F), jnp.float32),
            pltpu.VMEM((_OP, _NSCP), jnp.float32),
            pltpu.SemaphoreType.DMA,
            pltpu.SemaphoreType.DMA,
            pltpu.SemaphoreType.DMA,
        ],
    )(inputs, idx)
    return out[:, :, :, :_O, :_NSC]


# speedup vs baseline: 3.0414x; 1.2909x over previous
import jax
import jax.numpy as jnp
from jax import lax
from jax.experimental import pallas as pl
from jax.experimental.pallas import tpu as pltpu, tpu_sc as plsc

_B, _T, _S, _O, _F = 16, 4, 2, 14, 4096
_NSC = 3276
_OP = 16
_NSCP = 3328
_NTILE = _NSCP // 128
_NSLAB = _B * _T * _S
_NUNIT = 2 * _NSLAB
_NW = 32
_UNITS_PER_W = _NUNIT // _NW
_LANES = 16
_NJ = _NSCP // _LANES
_NIDX = _NJ * _LANES


def _gather_body(in_hbm, idx_hbm, out_hbm, idx_v, in_v0, in_v1,
                 out_v0, out_v1, isem0, isem1, osem0, osem1):
    c = lax.axis_index("c")
    s = lax.axis_index("s")
    wid = s * 2 + c
    unit0 = wid * _UNITS_PER_W
    pltpu.sync_copy(idx_hbm, idx_v)

    in_bufs = (in_v0, in_v1)
    out_bufs = (out_v0, out_v1)
    in_sems = (isem0, isem1)
    out_sems = (osem0, osem1)

    def unit_idx(k):
        n = (unit0 + k) // 2
        tr = (unit0 + k) % 2
        return n // (_T * _S), (n // _S) % _T, n % _S, tr

    def start_in(k):
        b, t, ss, tr = unit_idx(k)
        return pltpu.async_copy(
            in_hbm.at[b, t, ss, pl.ds(tr * 8, 8)],
            in_bufs[k % 2],
            in_sems[k % 2],
        )

    def start_out(k):
        b, t, ss, tr = unit_idx(k)
        out_v = out_bufs[k % 2]
        copies = []
        for tc in range(_NTILE):
            copies.append(
                pltpu.async_copy(
                    out_v.at[tc],
                    out_hbm.at[b, t, ss, pl.ds(tr * 8, 8),
                               pl.ds(tc * 128, 128)],
                    out_sems[k % 2],
                )
            )
        return copies

    def gather_unit(in_v, out_v):
        @plsc.parallel_loop(0, _NJ, unroll=2)
        def j_body(j):
            idx = idx_v[pl.ds(j * _LANES, _LANES)]
            tc = j // 8
            l0 = (j % 8) * _LANES
            for r in range(8):
                out_v[tc, r, pl.ds(l0, _LANES)] = plsc.load_gather(
                    in_v, [jnp.full((_LANES,), r, jnp.int32), idx]
                )

    in_copies = [None] * _UNITS_PER_W
    out_copies = [None] * _UNITS_PER_W
    in_copies[0] = start_in(0)
    for k in range(_UNITS_PER_W):
        if k + 1 < _UNITS_PER_W:
            in_copies[k + 1] = start_in(k + 1)
        in_copies[k].wait()
        if k >= 2:
            for cp in out_copies[k - 2]:
                cp.wait()
        gather_unit(in_bufs[k % 2], out_bufs[k % 2])
        out_copies[k] = start_out(k)
    for cp in out_copies[_UNITS_PER_W - 2]:
        cp.wait()
    for cp in out_copies[_UNITS_PER_W - 1]:
        cp.wait()


def kernel(inputs, sc_ind):
    x = jnp.pad(inputs, ((0, 0), (0, 0), (0, 0), (0, _OP - _O), (0, 0)))
    idx32 = sc_ind.astype(jnp.int32)
    idx = jnp.concatenate(
        [idx32, jnp.full((_NIDX - _NSC,), idx32[-1], jnp.int32)]
    )
    mesh = plsc.VectorSubcoreMesh(core_axis_name="c", subcore_axis_name="s")
    out = pl.kernel(
        _gather_body,
        mesh=mesh,
        compiler_params=pltpu.CompilerParams(
            needs_layout_passes=False, use_tc_tiling_on_sc=True
        ),
        out_type=jax.ShapeDtypeStruct((_B, _T, _S, _OP, _NSCP), jnp.float32),
        scratch_types=[
            pltpu.VMEM((_NIDX,), jnp.int32),
            pltpu.VMEM((8, _F), jnp.float32),
            pltpu.VMEM((8, _F), jnp.float32),
            pltpu.VMEM((_NTILE, 8, 128), jnp.float32),
            pltpu.VMEM((_NTILE, 8, 128), jnp.float32),
            pltpu.SemaphoreType.DMA,
            pltpu.SemaphoreType.DMA,
            pltpu.SemaphoreType.DMA,
            pltpu.SemaphoreType.DMA,
        ],
    )(x, idx)
    return out[:, :, :, :_O, :_NSC]


# speedup vs baseline: 3.5878x; 1.1796x over previous
import jax
import jax.numpy as jnp
from jax import lax
from jax.experimental import pallas as pl
from jax.experimental.pallas import tpu as pltpu, tpu_sc as plsc

_B, _T, _S, _O, _F = 16, 4, 2, 14, 4096
_NSC = 3276
_NSCP = 3328
_NTILE = _NSCP // 128
_NSLAB = _B * _T * _S
_NW = 32
_SLABS_PER_W = _NSLAB // _NW
_LANES = 16
_NJ = _NSCP // _LANES
_NIDX = _NJ * _LANES
_ROWS_TR = (8, _O - 8)


def _gather_body(in_hbm, idx_hbm, out_hbm, idx_v, in_a, in_b, out_v,
                 isem0, isem1, osem):
    c = lax.axis_index("c")
    s = lax.axis_index("s")
    wid = s * 2 + c
    slab0 = wid * _SLABS_PER_W
    pltpu.sync_copy(idx_hbm, idx_v)

    in_bufs = (in_a, in_b)
    in_sems = (isem0, isem1)

    def slab_idx(k):
        n = slab0 + k
        return n // (_T * _S), (n // _S) % _T, n % _S

    def start_in(ch):
        b, t, ss = slab_idx(ch // 2)
        tr = ch % 2
        return pltpu.async_copy(
            in_hbm.at[b, t, ss, pl.ds(tr * 8, _ROWS_TR[tr])],
            in_bufs[tr],
            in_sems[tr],
        )

    def start_out(k):
        b, t, ss = slab_idx(k)
        copies = []
        for tr in range(2):
            rows = _ROWS_TR[tr]
            for tc in range(_NTILE):
                copies.append(
                    pltpu.async_copy(
                        out_v.at[tr, tc, pl.ds(0, rows)],
                        out_hbm.at[b, t, ss, pl.ds(tr * 8, rows),
                                   pl.ds(tc * 128, 128)],
                        osem,
                    )
                )
        return copies

    def gather_chunk(in_v, tr):
        @plsc.parallel_loop(0, _NJ, unroll=2)
        def j_body(j):
            idx = idx_v[pl.ds(j * _LANES, _LANES)]
            tc = j // 8
            l0 = (j % 8) * _LANES
            for r in range(_ROWS_TR[tr]):
                out_v[tr, tc, r, pl.ds(l0, _LANES)] = plsc.load_gather(
                    in_v, [jnp.full((_LANES,), r, jnp.int32), idx]
                )

    in_copies = [None] * (2 * _SLABS_PER_W)
    out_copies = None
    in_copies[0] = start_in(0)
    for k in range(_SLABS_PER_W):
        for tr in range(2):
            ch = 2 * k + tr
            if ch + 1 < 2 * _SLABS_PER_W:
                in_copies[ch + 1] = start_in(ch + 1)
            in_copies[ch].wait()
            if tr == 0 and out_copies is not None:
                for cp in out_copies:
                    cp.wait()
            gather_chunk(in_bufs[tr], tr)
        out_copies = start_out(k)
    for cp in out_copies:
        cp.wait()


def kernel(inputs, sc_ind):
    idx32 = sc_ind.astype(jnp.int32)
    idx = jnp.concatenate(
        [idx32, jnp.full((_NIDX - _NSC,), idx32[-1], jnp.int32)]
    )
    mesh = plsc.VectorSubcoreMesh(core_axis_name="c", subcore_axis_name="s")
    out = pl.kernel(
        _gather_body,
        mesh=mesh,
        compiler_params=pltpu.CompilerParams(
            needs_layout_passes=False, use_tc_tiling_on_sc=True
        ),
        out_type=jax.ShapeDtypeStruct((_B, _T, _S, _O, _NSCP), jnp.float32),
        scratch_types=[
            pltpu.VMEM((_NIDX,), jnp.int32),
            pltpu.VMEM((8, _F), jnp.float32),
            pltpu.VMEM((_O - 8, _F), jnp.float32),
            pltpu.VMEM((2, _NTILE, 8, 128), jnp.float32),
            pltpu.SemaphoreType.DMA,
            pltpu.SemaphoreType.DMA,
            pltpu.SemaphoreType.DMA,
        ],
    )(inputs, idx)
    return out[..., :_NSC]
